# baseline (speedup 1.0000x reference)
import functools

import jax
import jax.numpy as jnp
from jax import lax
from jax.experimental import pallas as pl
from jax.experimental.pallas import tpu as pltpu
from jax.experimental.pallas import tpu_sc as plsc

BATCH = 16384
EMB = 64
NROWS = 1000000
IDX_CHUNK = 128
TBLK = 4096


def _transform_block(ut_ref, it_ref, ubf_ref, ibf_ref):
    eye = (lax.broadcasted_iota(jnp.int32, (EMB, EMB), 0)
           == lax.broadcasted_iota(jnp.int32, (EMB, EMB), 1)
           ).astype(jnp.bfloat16)
    for src, dst in ((ut_ref, ubf_ref), (it_ref, ibf_ref)):
        xb = src[...].astype(jnp.bfloat16)
        t = lax.dot_general(xb, eye, (((0,), (0,)), ((), ())),
                            preferred_element_type=jnp.float32)
        dst[...] = t.astype(jnp.bfloat16)


def _to_bf16_rows(user_emb, item_emb):
    ut = user_emb.T
    it = item_emb.T
    grid = (pl.cdiv(NROWS, TBLK),)
    return pl.pallas_call(
        _transform_block,
        grid=grid,
        in_specs=[
            pl.BlockSpec((EMB, TBLK), lambda j: (0, j)),
            pl.BlockSpec((EMB, TBLK), lambda j: (0, j)),
        ],
        out_specs=[
            pl.BlockSpec((TBLK, EMB), lambda j: (j, 0)),
            pl.BlockSpec((TBLK, EMB), lambda j: (j, 0)),
        ],
        out_shape=[
            jax.ShapeDtypeStruct((NROWS, EMB), jnp.bfloat16),
            jax.ShapeDtypeStruct((NROWS, EMB), jnp.bfloat16),
        ],
    )(ut, it)


def _sc_gather(user_idx2d, item_idx2d, user_emb, item_emb):
    info = plsc.get_sparse_core_info()
    nc, ns = info.num_cores, info.num_subcores
    nw = nc * ns
    rows_per_w = BATCH // nw
    chunks_per_w = rows_per_w // IDX_CHUNK

    mesh = plsc.VectorSubcoreMesh(core_axis_name="c", subcore_axis_name="s")

    @functools.partial(
        pl.kernel,
        mesh=mesh,
        compiler_params=pltpu.CompilerParams(use_tc_tiling_on_sc=False),
        out_type=[
            jax.ShapeDtypeStruct((BATCH, EMB), jnp.bfloat16),
            jax.ShapeDtypeStruct((BATCH, EMB), jnp.bfloat16),
        ],
        scratch_types=[
            pltpu.VMEM((chunks_per_w, IDX_CHUNK), jnp.int32),
            pltpu.VMEM((chunks_per_w, IDX_CHUNK), jnp.int32),
            pltpu.VMEM((rows_per_w, EMB), jnp.bfloat16),
            pltpu.VMEM((rows_per_w, EMB), jnp.bfloat16),
            pltpu.SemaphoreType.DMA,
        ],
    )
    def gather_k(uidx_hbm, iidx_hbm, uemb_hbm, iemb_hbm, out_u, out_i,
                 uidx_v, iidx_v, urows_v, irows_v, sem):
        wid = lax.axis_index("s") * nc + lax.axis_index("c")
        crow = wid * chunks_per_w
        pltpu.sync_copy(uidx_hbm.at[pl.ds(crow, chunks_per_w)], uidx_v)
        pltpu.sync_copy(iidx_hbm.at[pl.ds(crow, chunks_per_w)], iidx_v)
        cps = []
        for j in range(chunks_per_w):
            cps.append(pltpu.async_copy(
                uemb_hbm.at[uidx_v.at[j]],
                urows_v.at[pl.ds(j * IDX_CHUNK, IDX_CHUNK)], sem))
            cps.append(pltpu.async_copy(
                iemb_hbm.at[iidx_v.at[j]],
                irows_v.at[pl.ds(j * IDX_CHUNK, IDX_CHUNK)], sem))
        for cp in cps:
            cp.wait()
        base = wid * rows_per_w
        pltpu.sync_copy(urows_v, out_u.at[pl.ds(base, rows_per_w)])
        pltpu.sync_copy(irows_v, out_i.at[pl.ds(base, rows_per_w)])

    return gather_k(user_idx2d, item_idx2d, user_emb, item_emb)


def _mlp_block(u_ref, v_ref, w1u_ref, w1i_ref, b1_ref, w2_ref, b2_ref,
               w3_ref, b3_ref, wo_ref, bo_ref, out_ref):
    x = jnp.dot(u_ref[...], w1u_ref[...], preferred_element_type=jnp.float32)
    x += jnp.dot(v_ref[...], w1i_ref[...], preferred_element_type=jnp.float32)
    x = jnp.maximum(x + b1_ref[...], 0.0).astype(jnp.bfloat16)
    x = jnp.dot(x, w2_ref[...], preferred_element_type=jnp.float32)
    x = jnp.maximum(x + b2_ref[...], 0.0).astype(jnp.bfloat16)
    x = jnp.dot(x, w3_ref[...], preferred_element_type=jnp.float32)
    x = jnp.maximum(x + b3_ref[...], 0.0)
    logit = jnp.sum(x * wo_ref[...], axis=1) + bo_ref[0, 0]
    out_ref[...] = jax.nn.sigmoid(logit)


def kernel(user_indices, item_indices, user_emb, item_emb,
           W1, b1, W2, b2, W3, b3, Wo, bo):
    ubf, ibf = _to_bf16_rows(user_emb, item_emb)
    uidx2d = user_indices.reshape(BATCH // IDX_CHUNK, IDX_CHUNK)
    iidx2d = item_indices.reshape(BATCH // IDX_CHUNK, IDX_CHUNK)
    u_rows, i_rows = _sc_gather(uidx2d, iidx2d, ubf, ibf)

    w1u = W1[:, :EMB].T.astype(jnp.bfloat16)
    w1i = W1[:, EMB:].T.astype(jnp.bfloat16)
    w2t = W2.T.astype(jnp.bfloat16)
    w3t = W3.T.astype(jnp.bfloat16)
    b1r = b1.reshape(1, -1)
    b2r = b2.reshape(1, -1)
    b3r = b3.reshape(1, -1)
    wor = Wo.reshape(1, -1)
    bor = bo.reshape(1, 1)

    bb = 2048
    grid = (BATCH // bb,)
    full = lambda i: (0, 0)
    out = pl.pallas_call(
        _mlp_block,
        grid=grid,
        in_specs=[
            pl.BlockSpec((bb, EMB), lambda i: (i, 0)),
            pl.BlockSpec((bb, EMB), lambda i: (i, 0)),
            pl.BlockSpec(w1u.shape, full),
            pl.BlockSpec(w1i.shape, full),
            pl.BlockSpec(b1r.shape, full),
            pl.BlockSpec(w2t.shape, full),
            pl.BlockSpec(b2r.shape, full),
            pl.BlockSpec(w3t.shape, full),
            pl.BlockSpec(b3r.shape, full),
            pl.BlockSpec(wor.shape, full),
            pl.BlockSpec(bor.shape, full),
        ],
        out_specs=pl.BlockSpec((bb,), lambda i: (i,)),
        out_shape=jax.ShapeDtypeStruct((BATCH,), jnp.float32),
    )(u_rows, i_rows, w1u, w1i, b1r, w2t, b2r, w3t, b3r, wor, bor)
    return out


# speedup vs baseline: 3.4879x; 3.4879x over previous
import functools

import jax
import jax.numpy as jnp
from jax import lax
from jax.experimental import pallas as pl
from jax.experimental.pallas import tpu as pltpu
from jax.experimental.pallas import tpu_sc as plsc

BATCH = 16384
EMB = 64
NROWS = 1000000
IDX_CHUNK = 128
TBLK = 8192


def _transform_block(ut_ref, it_ref, out_ref):
    eye = (lax.broadcasted_iota(jnp.int32, (EMB, EMB), 0)
           == lax.broadcasted_iota(jnp.int32, (EMB, EMB), 1)
           ).astype(jnp.bfloat16)
    xu = ut_ref[...].astype(jnp.bfloat16)
    tu = lax.dot_general(xu, eye, (((0,), (0,)), ((), ())),
                         preferred_element_type=jnp.float32)
    out_ref[:, 0:EMB] = tu
    xi = it_ref[...].astype(jnp.bfloat16)
    ti = lax.dot_general(xi, eye, (((0,), (0,)), ((), ())),
                         preferred_element_type=jnp.float32)
    out_ref[:, EMB:2 * EMB] = ti


def _to_stacked(user_emb, item_emb):
    ut = user_emb.T
    it = item_emb.T
    grid = (pl.cdiv(NROWS, TBLK),)
    return pl.pallas_call(
        _transform_block,
        grid=grid,
        in_specs=[
            pl.BlockSpec((EMB, TBLK), lambda j: (0, j)),
            pl.BlockSpec((EMB, TBLK), lambda j: (0, j)),
        ],
        out_specs=pl.BlockSpec((TBLK, 2 * EMB), lambda j: (j, 0)),
        out_shape=jax.ShapeDtypeStruct((NROWS, 2 * EMB), jnp.float32),
    )(ut, it)


def _sc_gather(user_idx2d, item_idx2d, stacked):
    info = plsc.get_sparse_core_info()
    nc, ns = info.num_cores, info.num_subcores
    nw = nc * ns
    rows_per_w = BATCH // nw
    chunks_per_w = rows_per_w // IDX_CHUNK

    mesh = plsc.VectorSubcoreMesh(core_axis_name="c", subcore_axis_name="s")

    @functools.partial(
        pl.kernel,
        mesh=mesh,
        out_type=[
            jax.ShapeDtypeStruct((BATCH // IDX_CHUNK, IDX_CHUNK, 2 * EMB),
                                 jnp.float32),
            jax.ShapeDtypeStruct((BATCH // IDX_CHUNK, IDX_CHUNK, 2 * EMB),
                                 jnp.float32),
        ],
        scratch_types=[
            pltpu.VMEM((2 * chunks_per_w, IDX_CHUNK), jnp.int32),
            pltpu.VMEM((chunks_per_w, IDX_CHUNK, 2 * EMB), jnp.float32),
            pltpu.SemaphoreType.DMA,
        ],
    )
    def gather_k(uidx_hbm, iidx_hbm, tbl_hbm, out_u, out_i,
                 idx_v, rows_v, sem):
        wid = lax.axis_index("s") * nc + lax.axis_index("c")
        crow = wid * chunks_per_w
        pltpu.sync_copy(uidx_hbm.at[pl.ds(crow, chunks_per_w)],
                        idx_v.at[pl.ds(0, chunks_per_w)])
        pltpu.sync_copy(iidx_hbm.at[pl.ds(crow, chunks_per_w)],
                        idx_v.at[pl.ds(chunks_per_w, chunks_per_w)])
        for half, out in enumerate((out_u, out_i)):
            cps = []
            for j in range(chunks_per_w):
                cps.append(pltpu.async_copy(
                    tbl_hbm.at[idx_v.at[half * chunks_per_w + j]],
                    rows_v.at[j], sem))
            for cp in cps:
                cp.wait()
            pltpu.sync_copy(rows_v, out.at[pl.ds(crow, chunks_per_w)])

    return gather_k(user_idx2d, item_idx2d, stacked)


def _mlp_block(u_ref, v_ref, w1u_ref, w1i_ref, b1_ref,
               w2_ref, b2_ref, w3_ref, b3_ref, wo_ref, bo_ref, out_ref):
    usel = u_ref[:, :EMB].astype(jnp.bfloat16)
    vsel = v_ref[:, EMB:].astype(jnp.bfloat16)
    x = jnp.dot(usel, w1u_ref[...], preferred_element_type=jnp.float32)
    x += jnp.dot(vsel, w1i_ref[...], preferred_element_type=jnp.float32)
    x = jnp.maximum(x + b1_ref[...], 0.0).astype(jnp.bfloat16)
    x = jnp.dot(x, w2_ref[...], preferred_element_type=jnp.float32)
    x = jnp.maximum(x + b2_ref[...], 0.0).astype(jnp.bfloat16)
    x = jnp.dot(x, w3_ref[...], preferred_element_type=jnp.float32)
    x = jnp.maximum(x + b3_ref[...], 0.0)
    logit = jnp.sum(x * wo_ref[...], axis=1) + bo_ref[0, 0]
    out_ref[...] = jax.nn.sigmoid(logit)


def kernel(user_indices, item_indices, user_emb, item_emb,
           W1, b1, W2, b2, W3, b3, Wo, bo):
    stacked = _to_stacked(user_emb, item_emb)
    uidx2d = user_indices.reshape(BATCH // IDX_CHUNK, IDX_CHUNK)
    iidx2d = item_indices.reshape(BATCH // IDX_CHUNK, IDX_CHUNK)
    u_rows3, i_rows3 = _sc_gather(uidx2d, iidx2d, stacked)
    u_rows = u_rows3.reshape(BATCH, 2 * EMB)
    i_rows = i_rows3.reshape(BATCH, 2 * EMB)

    w1u = W1[:, :EMB].T.astype(jnp.bfloat16)
    w1i = W1[:, EMB:].T.astype(jnp.bfloat16)
    w2t = W2.T.astype(jnp.bfloat16)
    w3t = W3.T.astype(jnp.bfloat16)
    b1r = b1.reshape(1, -1)
    b2r = b2.reshape(1, -1)
    b3r = b3.reshape(1, -1)
    wor = Wo.reshape(1, -1)
    bor = bo.reshape(1, 1)

    bb = 2048
    grid = (BATCH // bb,)
    full = lambda i: (0, 0)
    out = pl.pallas_call(
        _mlp_block,
        grid=grid,
        in_specs=[
            pl.BlockSpec((bb, 2 * EMB), lambda i: (i, 0)),
            pl.BlockSpec((bb, 2 * EMB), lambda i: (i, 0)),
            pl.BlockSpec(w1u.shape, full),
            pl.BlockSpec(w1i.shape, full),
            pl.BlockSpec(b1r.shape, full),
            pl.BlockSpec(w2t.shape, full),
            pl.BlockSpec(b2r.shape, full),
            pl.BlockSpec(w3t.shape, full),
            pl.BlockSpec(b3r.shape, full),
            pl.BlockSpec(wor.shape, full),
            pl.BlockSpec(bor.shape, full),
        ],
        out_specs=pl.BlockSpec((bb,), lambda i: (i,)),
        out_shape=jax.ShapeDtypeStruct((BATCH,), jnp.float32),
    )(u_rows, i_rows, w1u, w1i, b1r, w2t, b2r, w3t, b3r, wor, bor)
    return out


# speedup vs baseline: 4.1484x; 1.1894x over previous
import functools

import jax
import jax.numpy as jnp
from jax import lax
from jax.experimental import pallas as pl
from jax.experimental.pallas import tpu as pltpu
from jax.experimental.pallas import tpu_sc as plsc

BATCH = 16384
EMB = 64
NROWS = 1000000
IDX_CHUNK = 128
TBLK = 8192
HBLK = TBLK // 2
NBLK = (NROWS + TBLK - 1) // TBLK
NPACK = NBLK * HBLK


def _transform_block(ut_ref, it_ref, out_ref):
    eye = (lax.broadcasted_iota(jnp.int32, (EMB, EMB), 0)
           == lax.broadcasted_iota(jnp.int32, (EMB, EMB), 1)
           ).astype(jnp.bfloat16)
    xu = ut_ref[...].astype(jnp.bfloat16)
    tu = lax.dot_general(xu, eye, (((0,), (0,)), ((), ())),
                         preferred_element_type=jnp.float32)
    xi = it_ref[...].astype(jnp.bfloat16)
    ti = lax.dot_general(xi, eye, (((0,), (0,)), ((), ())),
                         preferred_element_type=jnp.float32)
    iu = lax.bitcast_convert_type(tu, jnp.int32)
    ii = lax.bitcast_convert_type(ti, jnp.int32)
    packed = (iu & jnp.int32(-65536)) | ((ii >> 16) & jnp.int32(0xFFFF))
    out_ref[:, 0:EMB] = packed[0:HBLK]
    out_ref[:, EMB:2 * EMB] = packed[HBLK:TBLK]


def _to_packed(user_emb, item_emb):
    ut = user_emb.T
    it = item_emb.T
    grid = (pl.cdiv(NROWS, TBLK),)
    return pl.pallas_call(
        _transform_block,
        grid=grid,
        in_specs=[
            pl.BlockSpec((EMB, TBLK), lambda j: (0, j)),
            pl.BlockSpec((EMB, TBLK), lambda j: (0, j)),
        ],
        out_specs=pl.BlockSpec((HBLK, 2 * EMB), lambda j: (j, 0)),
        out_shape=jax.ShapeDtypeStruct((NPACK, 2 * EMB), jnp.int32),
    )(ut, it)


def _sc_gather(user_idx2d, item_idx2d, packed):
    info = plsc.get_sparse_core_info()
    nc, ns = info.num_cores, info.num_subcores
    nw = nc * ns
    rows_per_w = BATCH // nw
    chunks_per_w = rows_per_w // IDX_CHUNK

    mesh = plsc.VectorSubcoreMesh(core_axis_name="c", subcore_axis_name="s")

    @functools.partial(
        pl.kernel,
        mesh=mesh,
        out_type=[
            jax.ShapeDtypeStruct((BATCH // IDX_CHUNK, IDX_CHUNK, 2 * EMB),
                                 jnp.int32),
            jax.ShapeDtypeStruct((BATCH // IDX_CHUNK, IDX_CHUNK, 2 * EMB),
                                 jnp.int32),
        ],
        scratch_types=[
            pltpu.VMEM((2 * chunks_per_w, IDX_CHUNK), jnp.int32),
            pltpu.VMEM((chunks_per_w, IDX_CHUNK, 2 * EMB), jnp.int32),
            pltpu.SemaphoreType.DMA,
        ],
    )
    def gather_k(uidx_hbm, iidx_hbm, tbl_hbm, out_u, out_i,
                 idx_v, rows_v, sem):
        wid = lax.axis_index("s") * nc + lax.axis_index("c")
        crow = wid * chunks_per_w
        pltpu.sync_copy(uidx_hbm.at[pl.ds(crow, chunks_per_w)],
                        idx_v.at[pl.ds(0, chunks_per_w)])
        pltpu.sync_copy(iidx_hbm.at[pl.ds(crow, chunks_per_w)],
                        idx_v.at[pl.ds(chunks_per_w, chunks_per_w)])
        for half, out in enumerate((out_u, out_i)):
            cps = []
            for j in range(chunks_per_w):
                cps.append(pltpu.async_copy(
                    tbl_hbm.at[idx_v.at[half * chunks_per_w + j]],
                    rows_v.at[j], sem))
            for cp in cps:
                cp.wait()
            pltpu.sync_copy(rows_v, out.at[pl.ds(crow, chunks_per_w)])

    return gather_k(user_idx2d, item_idx2d, packed)


def _mlp_block(u_ref, v_ref, uh_ref, vh_ref, w1u_ref, w1i_ref, b1_ref,
               w2_ref, b2_ref, w3_ref, b3_ref, wo_ref, bo_ref, out_ref):
    usel = jnp.where(uh_ref[...] == 1, u_ref[:, EMB:], u_ref[:, :EMB])
    vsel = jnp.where(vh_ref[...] == 1, v_ref[:, EMB:], v_ref[:, :EMB])
    uval = lax.bitcast_convert_type(usel & jnp.int32(-65536),
                                    jnp.float32).astype(jnp.bfloat16)
    vval = lax.bitcast_convert_type(lax.shift_left(vsel, 16),
                                    jnp.float32).astype(jnp.bfloat16)
    x = jnp.dot(uval, w1u_ref[...], preferred_element_type=jnp.float32)
    x += jnp.dot(vval, w1i_ref[...], preferred_element_type=jnp.float32)
    x = jnp.maximum(x + b1_ref[...], 0.0).astype(jnp.bfloat16)
    x = jnp.dot(x, w2_ref[...], preferred_element_type=jnp.float32)
    x = jnp.maximum(x + b2_ref[...], 0.0).astype(jnp.bfloat16)
    x = jnp.dot(x, w3_ref[...], preferred_element_type=jnp.float32)
    x = jnp.maximum(x + b3_ref[...], 0.0)
    logit = jnp.sum(x * wo_ref[...], axis=1) + bo_ref[0, 0]
    out_ref[...] = jax.nn.sigmoid(logit)


def _pack_index(r):
    return ((r >> 13) << 12) | (r & (HBLK - 1)), (r >> 12) & 1


def kernel(user_indices, item_indices, user_emb, item_emb,
           W1, b1, W2, b2, W3, b3, Wo, bo):
    packed = _to_packed(user_emb, item_emb)
    up, uh = _pack_index(user_indices)
    ip, ih = _pack_index(item_indices)
    uidx2d = up.reshape(BATCH // IDX_CHUNK, IDX_CHUNK)
    iidx2d = ip.reshape(BATCH // IDX_CHUNK, IDX_CHUNK)
    u_rows3, i_rows3 = _sc_gather(uidx2d, iidx2d, packed)
    u_rows = u_rows3.reshape(BATCH, 2 * EMB)
    i_rows = i_rows3.reshape(BATCH, 2 * EMB)

    w1u = W1[:, :EMB].T.astype(jnp.bfloat16)
    w1i = W1[:, EMB:].T.astype(jnp.bfloat16)
    w2t = W2.T.astype(jnp.bfloat16)
    w3t = W3.T.astype(jnp.bfloat16)
    b1r = b1.reshape(1, -1)
    b2r = b2.reshape(1, -1)
    b3r = b3.reshape(1, -1)
    wor = Wo.reshape(1, -1)
    bor = bo.reshape(1, 1)

    bb = 2048
    grid = (BATCH // bb,)
    full = lambda i: (0, 0)
    out = pl.pallas_call(
        _mlp_block,
        grid=grid,
        in_specs=[
            pl.BlockSpec((bb, 2 * EMB), lambda i: (i, 0)),
            pl.BlockSpec((bb, 2 * EMB), lambda i: (i, 0)),
            pl.BlockSpec((bb, 1), lambda i: (i, 0)),
            pl.BlockSpec((bb, 1), lambda i: (i, 0)),
            pl.BlockSpec(w1u.shape, full),
            pl.BlockSpec(w1i.shape, full),
            pl.BlockSpec(b1r.shape, full),
            pl.BlockSpec(w2t.shape, full),
            pl.BlockSpec(b2r.shape, full),
            pl.BlockSpec(w3t.shape, full),
            pl.BlockSpec(b3r.shape, full),
            pl.BlockSpec(wor.shape, full),
            pl.BlockSpec(bor.shape, full),
        ],
        out_specs=pl.BlockSpec((bb,), lambda i: (i,)),
        out_shape=jax.ShapeDtypeStruct((BATCH,), jnp.float32),
    )(u_rows, i_rows, uh.reshape(BATCH, 1), ih.reshape(BATCH, 1),
      w1u, w1i, b1r, w2t, b2r, w3t, b3r, wor, bor)
    return out


# speedup vs baseline: 4.5868x; 1.1057x over previous
import functools

import jax
import jax.numpy as jnp
from jax import lax
from jax.experimental import pallas as pl
from jax.experimental.pallas import tpu as pltpu
from jax.experimental.pallas import tpu_sc as plsc

BATCH = 16384
EMB = 64
NROWS = 1000000
IDX_CHUNK = 128
TBLK = 16384
HBLK = TBLK // 2
NBLK = (NROWS + TBLK - 1) // TBLK
NPACK = NBLK * HBLK


def _transform_block(ut_ref, it_ref, out_ref):
    eye = (lax.broadcasted_iota(jnp.int32, (EMB, EMB), 0)
           == lax.broadcasted_iota(jnp.int32, (EMB, EMB), 1)
           ).astype(jnp.bfloat16)
    xu = ut_ref[...].astype(jnp.bfloat16)
    tu = lax.dot_general(xu, eye, (((0,), (0,)), ((), ())),
                         preferred_element_type=jnp.float32)
    xi = it_ref[...].astype(jnp.bfloat16)
    ti = lax.dot_general(xi, eye, (((0,), (0,)), ((), ())),
                         preferred_element_type=jnp.float32)
    iu = lax.bitcast_convert_type(tu, jnp.int32)
    ii = lax.bitcast_convert_type(ti, jnp.int32)
    packed = (iu & jnp.int32(-65536)) | ((ii >> 16) & jnp.int32(0xFFFF))
    out_ref[:, 0:EMB] = packed[0:HBLK]
    out_ref[:, EMB:2 * EMB] = packed[HBLK:TBLK]


def _to_packed(user_emb, item_emb):
    ut = user_emb.T
    it = item_emb.T
    grid = (pl.cdiv(NROWS, TBLK),)
    return pl.pallas_call(
        _transform_block,
        grid=grid,
        in_specs=[
            pl.BlockSpec((EMB, TBLK), lambda j: (0, j)),
            pl.BlockSpec((EMB, TBLK), lambda j: (0, j)),
        ],
        out_specs=pl.BlockSpec((HBLK, 2 * EMB), lambda j: (j, 0)),
        out_shape=jax.ShapeDtypeStruct((NPACK, 2 * EMB), jnp.int32),
    )(ut, it)


def _sc_gather(user_idx2d, item_idx2d, packed):
    info = plsc.get_sparse_core_info()
    nc, ns = info.num_cores, info.num_subcores
    nw = nc * ns
    rows_per_w = BATCH // nw
    chunks_per_w = rows_per_w // IDX_CHUNK

    mesh = plsc.VectorSubcoreMesh(core_axis_name="c", subcore_axis_name="s")

    @functools.partial(
        pl.kernel,
        mesh=mesh,
        out_type=[
            jax.ShapeDtypeStruct((BATCH // IDX_CHUNK, IDX_CHUNK, 2 * EMB),
                                 jnp.int32),
            jax.ShapeDtypeStruct((BATCH // IDX_CHUNK, IDX_CHUNK, 2 * EMB),
                                 jnp.int32),
        ],
        scratch_types=[
            pltpu.VMEM((2 * chunks_per_w, IDX_CHUNK), jnp.int32),
            pltpu.VMEM((chunks_per_w, IDX_CHUNK, 2 * EMB), jnp.int32),
            pltpu.SemaphoreType.DMA,
        ],
    )
    def gather_k(uidx_hbm, iidx_hbm, tbl_hbm, out_u, out_i,
                 idx_v, rows_v, sem):
        wid = lax.axis_index("s") * nc + lax.axis_index("c")
        crow = wid * chunks_per_w
        pltpu.sync_copy(uidx_hbm.at[pl.ds(crow, chunks_per_w)],
                        idx_v.at[pl.ds(0, chunks_per_w)])
        pltpu.sync_copy(iidx_hbm.at[pl.ds(crow, chunks_per_w)],
                        idx_v.at[pl.ds(chunks_per_w, chunks_per_w)])
        for half, out in enumerate((out_u, out_i)):
            cps = []
            for j in range(chunks_per_w):
                cps.append(pltpu.async_copy(
                    tbl_hbm.at[idx_v.at[half * chunks_per_w + j]],
                    rows_v.at[j], sem))
            for cp in cps:
                cp.wait()
            pltpu.sync_copy(rows_v, out.at[pl.ds(crow, chunks_per_w)])

    return gather_k(user_idx2d, item_idx2d, packed)


def _mlp_block(u_ref, v_ref, uh_ref, vh_ref, w1u_ref, w1i_ref, b1_ref,
               w2_ref, b2_ref, w3_ref, b3_ref, wo_ref, bo_ref, out_ref):
    usel = jnp.where(uh_ref[...] == 1, u_ref[:, EMB:], u_ref[:, :EMB])
    vsel = jnp.where(vh_ref[...] == 1, v_ref[:, EMB:], v_ref[:, :EMB])
    uval = lax.bitcast_convert_type(usel & jnp.int32(-65536),
                                    jnp.float32).astype(jnp.bfloat16)
    vval = lax.bitcast_convert_type(lax.shift_left(vsel, 16),
                                    jnp.float32).astype(jnp.bfloat16)
    x = jnp.dot(uval, w1u_ref[...], preferred_element_type=jnp.float32)
    x += jnp.dot(vval, w1i_ref[...], preferred_element_type=jnp.float32)
    x = jnp.maximum(x + b1_ref[...], 0.0).astype(jnp.bfloat16)
    x = jnp.dot(x, w2_ref[...], preferred_element_type=jnp.float32)
    x = jnp.maximum(x + b2_ref[...], 0.0).astype(jnp.bfloat16)
    x = jnp.dot(x, w3_ref[...], preferred_element_type=jnp.float32)
    x = jnp.maximum(x + b3_ref[...], 0.0)
    logit = jnp.sum(x * wo_ref[...], axis=1) + bo_ref[0, 0]
    out_ref[...] = jax.nn.sigmoid(logit)


def _pack_index(r):
    return ((r >> 13) << 12) | (r & (HBLK - 1)), (r >> 12) & 1


def kernel(user_indices, item_indices, user_emb, item_emb,
           W1, b1, W2, b2, W3, b3, Wo, bo):
    packed = _to_packed(user_emb, item_emb)
    up, uh = _pack_index(user_indices)
    ip, ih = _pack_index(item_indices)
    uidx2d = up.reshape(BATCH // IDX_CHUNK, IDX_CHUNK)
    iidx2d = ip.reshape(BATCH // IDX_CHUNK, IDX_CHUNK)
    u_rows3, i_rows3 = _sc_gather(uidx2d, iidx2d, packed)
    u_rows = u_rows3.reshape(BATCH, 2 * EMB)
    i_rows = i_rows3.reshape(BATCH, 2 * EMB)

    w1u = W1[:, :EMB].T.astype(jnp.bfloat16)
    w1i = W1[:, EMB:].T.astype(jnp.bfloat16)
    w2t = W2.T.astype(jnp.bfloat16)
    w3t = W3.T.astype(jnp.bfloat16)
    b1r = b1.reshape(1, -1)
    b2r = b2.reshape(1, -1)
    b3r = b3.reshape(1, -1)
    wor = Wo.reshape(1, -1)
    bor = bo.reshape(1, 1)

    bb = 2048
    grid = (BATCH // bb,)
    full = lambda i: (0, 0)
    out = pl.pallas_call(
        _mlp_block,
        grid=grid,
        in_specs=[
            pl.BlockSpec((bb, 2 * EMB), lambda i: (i, 0)),
            pl.BlockSpec((bb, 2 * EMB), lambda i: (i, 0)),
            pl.BlockSpec((bb, 1), lambda i: (i, 0)),
            pl.BlockSpec((bb, 1), lambda i: (i, 0)),
            pl.BlockSpec(w1u.shape, full),
            pl.BlockSpec(w1i.shape, full),
            pl.BlockSpec(b1r.shape, full),
            pl.BlockSpec(w2t.shape, full),
            pl.BlockSpec(b2r.shape, full),
            pl.BlockSpec(w3t.shape, full),
            pl.BlockSpec(b3r.shape, full),
            pl.BlockSpec(wor.shape, full),
            pl.BlockSpec(bor.shape, full),
        ],
        out_specs=pl.BlockSpec((bb,), lambda i: (i,)),
        out_shape=jax.ShapeDtypeStruct((BATCH,), jnp.float32),
    )(u_rows, i_rows, uh.reshape(BATCH, 1), ih.reshape(BATCH, 1),
      w1u, w1i, b1r, w2t, b2r, w3t, b3r, wor, bor)
    return out


# speedup vs baseline: 4.5907x; 1.0009x over previous
import functools

import jax
import jax.numpy as jnp
from jax import lax
from jax.experimental import pallas as pl
from jax.experimental.pallas import tpu as pltpu
from jax.experimental.pallas import tpu_sc as plsc

BATCH = 16384
EMB = 64
NROWS = 1000000
IDX_CHUNK = 128
TBLK = 16384
HBLK = TBLK // 2
NBLK = (NROWS + TBLK - 1) // TBLK
NPACK = NBLK * HBLK


def _transform_block(ut_ref, it_ref, out_ref):
    eye = (lax.broadcasted_iota(jnp.int32, (EMB, EMB), 0)
           == lax.broadcasted_iota(jnp.int32, (EMB, EMB), 1)
           ).astype(jnp.bfloat16)
    xu = ut_ref[...].astype(jnp.bfloat16)
    tu = lax.dot_general(xu, eye, (((0,), (0,)), ((), ())),
                         preferred_element_type=jnp.float32)
    xi = it_ref[...].astype(jnp.bfloat16)
    ti = lax.dot_general(xi, eye, (((0,), (0,)), ((), ())),
                         preferred_element_type=jnp.float32)
    iu = lax.bitcast_convert_type(tu, jnp.int32)
    ii = lax.bitcast_convert_type(ti, jnp.int32)
    packed = (iu & jnp.int32(-65536)) | ((ii >> 16) & jnp.int32(0xFFFF))
    out_ref[:, 0:EMB] = packed[0:HBLK]
    out_ref[:, EMB:2 * EMB] = packed[HBLK:TBLK]


def _to_packed(user_emb, item_emb):
    ut = user_emb.T
    it = item_emb.T
    grid = (pl.cdiv(NROWS, TBLK),)
    return pl.pallas_call(
        _transform_block,
        grid=grid,
        in_specs=[
            pl.BlockSpec((EMB, TBLK), lambda j: (0, j)),
            pl.BlockSpec((EMB, TBLK), lambda j: (0, j)),
        ],
        out_specs=pl.BlockSpec((HBLK, 2 * EMB), lambda j: (j, 0)),
        out_shape=jax.ShapeDtypeStruct((NPACK, 2 * EMB), jnp.int32),
    )(ut, it)


def _sc_gather(user_idx2d, item_idx2d, packed):
    info = plsc.get_sparse_core_info()
    nc, ns = info.num_cores, info.num_subcores
    nw = nc * ns
    rows_per_w = BATCH // nw
    chunks_per_w = rows_per_w // IDX_CHUNK

    mesh = plsc.VectorSubcoreMesh(core_axis_name="c", subcore_axis_name="s")

    @functools.partial(
        pl.kernel,
        mesh=mesh,
        out_type=[
            jax.ShapeDtypeStruct((BATCH // IDX_CHUNK, IDX_CHUNK, 2 * EMB),
                                 jnp.int32),
            jax.ShapeDtypeStruct((BATCH // IDX_CHUNK, IDX_CHUNK, 2 * EMB),
                                 jnp.int32),
        ],
        scratch_types=[
            pltpu.VMEM((2 * chunks_per_w, IDX_CHUNK), jnp.int32),
            pltpu.VMEM((chunks_per_w, IDX_CHUNK, 2 * EMB), jnp.int32),
            pltpu.SemaphoreType.DMA,
        ],
    )
    def gather_k(uidx_hbm, iidx_hbm, tbl_hbm, out_u, out_i,
                 idx_v, rows_v, sem):
        wid = lax.axis_index("s") * nc + lax.axis_index("c")
        crow = wid * chunks_per_w
        pltpu.sync_copy(uidx_hbm.at[pl.ds(crow, chunks_per_w)],
                        idx_v.at[pl.ds(0, chunks_per_w)])
        pltpu.sync_copy(iidx_hbm.at[pl.ds(crow, chunks_per_w)],
                        idx_v.at[pl.ds(chunks_per_w, chunks_per_w)])
        for half, out in enumerate((out_u, out_i)):
            cps = []
            for j in range(chunks_per_w):
                cps.append(pltpu.async_copy(
                    tbl_hbm.at[idx_v.at[half * chunks_per_w + j]],
                    rows_v.at[j], sem))
            for cp in cps:
                cp.wait()
            pltpu.sync_copy(rows_v, out.at[pl.ds(crow, chunks_per_w)])

    return gather_k(user_idx2d, item_idx2d, packed)


def _mlp_block(u_ref, v_ref, uh_ref, vh_ref, w1u_ref, w1i_ref, b1_ref,
               w2_ref, b2_ref, w3_ref, b3_ref, wo_ref, bo_ref, out_ref):
    usel = jnp.where(uh_ref[...] == 1, u_ref[:, EMB:], u_ref[:, :EMB])
    vsel = jnp.where(vh_ref[...] == 1, v_ref[:, EMB:], v_ref[:, :EMB])
    uval = lax.bitcast_convert_type(usel & jnp.int32(-65536),
                                    jnp.float32).astype(jnp.bfloat16)
    vval = lax.bitcast_convert_type(lax.shift_left(vsel, 16),
                                    jnp.float32).astype(jnp.bfloat16)
    x = jnp.dot(uval, w1u_ref[...], preferred_element_type=jnp.float32)
    x += jnp.dot(vval, w1i_ref[...], preferred_element_type=jnp.float32)
    x = jnp.maximum(x + b1_ref[...], 0.0).astype(jnp.bfloat16)
    x = jnp.dot(x, w2_ref[...], preferred_element_type=jnp.float32)
    x = jnp.maximum(x + b2_ref[...], 0.0).astype(jnp.bfloat16)
    x = jnp.dot(x, w3_ref[...], preferred_element_type=jnp.float32)
    x = jnp.maximum(x + b3_ref[...], 0.0)
    logit = jnp.sum(x * wo_ref[...], axis=1) + bo_ref[0, 0]
    out_ref[...] = jax.nn.sigmoid(logit)


def _pack_index(r):
    return (r // TBLK) * HBLK + (r % HBLK), (r // HBLK) % 2


def kernel(user_indices, item_indices, user_emb, item_emb,
           W1, b1, W2, b2, W3, b3, Wo, bo):
    packed = _to_packed(user_emb, item_emb)
    up, uh = _pack_index(user_indices)
    ip, ih = _pack_index(item_indices)
    uidx2d = up.reshape(BATCH // IDX_CHUNK, IDX_CHUNK)
    iidx2d = ip.reshape(BATCH // IDX_CHUNK, IDX_CHUNK)
    u_rows3, i_rows3 = _sc_gather(uidx2d, iidx2d, packed)
    u_rows = u_rows3.reshape(BATCH, 2 * EMB)
    i_rows = i_rows3.reshape(BATCH, 2 * EMB)

    w1u = W1[:, :EMB].T.astype(jnp.bfloat16)
    w1i = W1[:, EMB:].T.astype(jnp.bfloat16)
    w2t = W2.T.astype(jnp.bfloat16)
    w3t = W3.T.astype(jnp.bfloat16)
    b1r = b1.reshape(1, -1)
    b2r = b2.reshape(1, -1)
    b3r = b3.reshape(1, -1)
    wor = Wo.reshape(1, -1)
    bor = bo.reshape(1, 1)

    bb = 2048
    grid = (BATCH // bb,)
    full = lambda i: (0, 0)
    out = pl.pallas_call(
        _mlp_block,
        grid=grid,
        in_specs=[
            pl.BlockSpec((bb, 2 * EMB), lambda i: (i, 0)),
            pl.BlockSpec((bb, 2 * EMB), lambda i: (i, 0)),
            pl.BlockSpec((bb, 1), lambda i: (i, 0)),
            pl.BlockSpec((bb, 1), lambda i: (i, 0)),
            pl.BlockSpec(w1u.shape, full),
            pl.BlockSpec(w1i.shape, full),
            pl.BlockSpec(b1r.shape, full),
            pl.BlockSpec(w2t.shape, full),
            pl.BlockSpec(b2r.shape, full),
            pl.BlockSpec(w3t.shape, full),
            pl.BlockSpec(b3r.shape, full),
            pl.BlockSpec(wor.shape, full),
            pl.BlockSpec(bor.shape, full),
        ],
        out_specs=pl.BlockSpec((bb,), lambda i: (i,)),
        out_shape=jax.ShapeDtypeStruct((BATCH,), jnp.float32),
    )(u_rows, i_rows, uh.reshape(BATCH, 1), ih.reshape(BATCH, 1),
      w1u, w1i, b1r, w2t, b2r, w3t, b3r, wor, bor)
    return out


# speedup vs baseline: 4.7928x; 1.0440x over previous
import functools

import jax
import jax.numpy as jnp
from jax import lax
from jax.experimental import pallas as pl
from jax.experimental.pallas import tpu as pltpu
from jax.experimental.pallas import tpu_sc as plsc

BATCH = 16384
EMB = 64
NROWS = 1000000
IDX_CHUNK = 128
TBLK = 32768
HBLK = TBLK // 2
NBLK = (NROWS + TBLK - 1) // TBLK
NPACK = NBLK * HBLK


def _transform_block(ut_ref, it_ref, out_ref):
    eye = (lax.broadcasted_iota(jnp.int32, (EMB, EMB), 0)
           == lax.broadcasted_iota(jnp.int32, (EMB, EMB), 1)
           ).astype(jnp.bfloat16)
    xu = ut_ref[...].astype(jnp.bfloat16)
    tu = lax.dot_general(xu, eye, (((0,), (0,)), ((), ())),
                         preferred_element_type=jnp.float32)
    xi = it_ref[...].astype(jnp.bfloat16)
    ti = lax.dot_general(xi, eye, (((0,), (0,)), ((), ())),
                         preferred_element_type=jnp.float32)
    iu = lax.bitcast_convert_type(tu, jnp.int32)
    ii = lax.bitcast_convert_type(ti, jnp.int32)
    packed = (iu & jnp.int32(-65536)) | ((ii >> 16) & jnp.int32(0xFFFF))
    out_ref[:, 0:EMB] = packed[0:HBLK]
    out_ref[:, EMB:2 * EMB] = packed[HBLK:TBLK]


def _to_packed(user_emb, item_emb):
    ut = user_emb.T
    it = item_emb.T
    grid = (pl.cdiv(NROWS, TBLK),)
    return pl.pallas_call(
        _transform_block,
        grid=grid,
        in_specs=[
            pl.BlockSpec((EMB, TBLK), lambda j: (0, j)),
            pl.BlockSpec((EMB, TBLK), lambda j: (0, j)),
        ],
        out_specs=pl.BlockSpec((HBLK, 2 * EMB), lambda j: (j, 0)),
        out_shape=jax.ShapeDtypeStruct((NPACK, 2 * EMB), jnp.int32),
    )(ut, it)


def _sc_gather(user_idx2d, item_idx2d, packed):
    info = plsc.get_sparse_core_info()
    nc, ns = info.num_cores, info.num_subcores
    nw = nc * ns
    rows_per_w = BATCH // nw
    chunks_per_w = rows_per_w // IDX_CHUNK

    mesh = plsc.VectorSubcoreMesh(core_axis_name="c", subcore_axis_name="s")

    @functools.partial(
        pl.kernel,
        mesh=mesh,
        out_type=[
            jax.ShapeDtypeStruct((BATCH // IDX_CHUNK, IDX_CHUNK, 2 * EMB),
                                 jnp.int32),
            jax.ShapeDtypeStruct((BATCH // IDX_CHUNK, IDX_CHUNK, 2 * EMB),
                                 jnp.int32),
        ],
        scratch_types=[
            pltpu.VMEM((2 * chunks_per_w, IDX_CHUNK), jnp.int32),
            pltpu.VMEM((chunks_per_w, IDX_CHUNK, 2 * EMB), jnp.int32),
            pltpu.SemaphoreType.DMA,
        ],
    )
    def gather_k(uidx_hbm, iidx_hbm, tbl_hbm, out_u, out_i,
                 idx_v, rows_v, sem):
        wid = lax.axis_index("s") * nc + lax.axis_index("c")
        crow = wid * chunks_per_w
        pltpu.sync_copy(uidx_hbm.at[pl.ds(crow, chunks_per_w)],
                        idx_v.at[pl.ds(0, chunks_per_w)])
        pltpu.sync_copy(iidx_hbm.at[pl.ds(crow, chunks_per_w)],
                        idx_v.at[pl.ds(chunks_per_w, chunks_per_w)])
        for half, out in enumerate((out_u, out_i)):
            cps = []
            for j in range(chunks_per_w):
                cps.append(pltpu.async_copy(
                    tbl_hbm.at[idx_v.at[half * chunks_per_w + j]],
                    rows_v.at[j], sem))
            for cp in cps:
                cp.wait()
            pltpu.sync_copy(rows_v, out.at[pl.ds(crow, chunks_per_w)])

    return gather_k(user_idx2d, item_idx2d, packed)


def _mlp_block(u_ref, v_ref, uh_ref, vh_ref, w1u_ref, w1i_ref, b1_ref,
               w2_ref, b2_ref, w3_ref, b3_ref, wo_ref, bo_ref, out_ref):
    usel = jnp.where(uh_ref[...] == 1, u_ref[:, EMB:], u_ref[:, :EMB])
    vsel = jnp.where(vh_ref[...] == 1, v_ref[:, EMB:], v_ref[:, :EMB])
    uval = lax.bitcast_convert_type(usel & jnp.int32(-65536),
                                    jnp.float32).astype(jnp.bfloat16)
    vval = lax.bitcast_convert_type(lax.shift_left(vsel, 16),
                                    jnp.float32).astype(jnp.bfloat16)
    x = jnp.dot(uval, w1u_ref[...], preferred_element_type=jnp.float32)
    x += jnp.dot(vval, w1i_ref[...], preferred_element_type=jnp.float32)
    x = jnp.maximum(x + b1_ref[...], 0.0).astype(jnp.bfloat16)
    x = jnp.dot(x, w2_ref[...], preferred_element_type=jnp.float32)
    x = jnp.maximum(x + b2_ref[...], 0.0).astype(jnp.bfloat16)
    x = jnp.dot(x, w3_ref[...], preferred_element_type=jnp.float32)
    x = jnp.maximum(x + b3_ref[...], 0.0)
    logit = jnp.sum(x * wo_ref[...], axis=1) + bo_ref[0, 0]
    out_ref[...] = jax.nn.sigmoid(logit)


def _pack_index(r):
    return (r // TBLK) * HBLK + (r % HBLK), (r // HBLK) % 2


def kernel(user_indices, item_indices, user_emb, item_emb,
           W1, b1, W2, b2, W3, b3, Wo, bo):
    packed = _to_packed(user_emb, item_emb)
    up, uh = _pack_index(user_indices)
    ip, ih = _pack_index(item_indices)
    uidx2d = up.reshape(BATCH // IDX_CHUNK, IDX_CHUNK)
    iidx2d = ip.reshape(BATCH // IDX_CHUNK, IDX_CHUNK)
    u_rows3, i_rows3 = _sc_gather(uidx2d, iidx2d, packed)
    u_rows = u_rows3.reshape(BATCH, 2 * EMB)
    i_rows = i_rows3.reshape(BATCH, 2 * EMB)

    w1u = W1[:, :EMB].T.astype(jnp.bfloat16)
    w1i = W1[:, EMB:].T.astype(jnp.bfloat16)
    w2t = W2.T.astype(jnp.bfloat16)
    w3t = W3.T.astype(jnp.bfloat16)
    b1r = b1.reshape(1, -1)
    b2r = b2.reshape(1, -1)
    b3r = b3.reshape(1, -1)
    wor = Wo.reshape(1, -1)
    bor = bo.reshape(1, 1)

    bb = 2048
    grid = (BATCH // bb,)
    full = lambda i: (0, 0)
    out = pl.pallas_call(
        _mlp_block,
        grid=grid,
        in_specs=[
            pl.BlockSpec((bb, 2 * EMB), lambda i: (i, 0)),
            pl.BlockSpec((bb, 2 * EMB), lambda i: (i, 0)),
            pl.BlockSpec((bb, 1), lambda i: (i, 0)),
            pl.BlockSpec((bb, 1), lambda i: (i, 0)),
            pl.BlockSpec(w1u.shape, full),
            pl.BlockSpec(w1i.shape, full),
            pl.BlockSpec(b1r.shape, full),
            pl.BlockSpec(w2t.shape, full),
            pl.BlockSpec(b2r.shape, full),
            pl.BlockSpec(w3t.shape, full),
            pl.BlockSpec(b3r.shape, full),
            pl.BlockSpec(wor.shape, full),
            pl.BlockSpec(bor.shape, full),
        ],
        out_specs=pl.BlockSpec((bb,), lambda i: (i,)),
        out_shape=jax.ShapeDtypeStruct((BATCH,), jnp.float32),
    )(u_rows, i_rows, uh.reshape(BATCH, 1), ih.reshape(BATCH, 1),
      w1u, w1i, b1r, w2t, b2r, w3t, b3r, wor, bor)
    return out


# speedup vs baseline: 4.8125x; 1.0041x over previous
import functools

import jax
import jax.numpy as jnp
from jax import lax
from jax.experimental import pallas as pl
from jax.experimental.pallas import tpu as pltpu
from jax.experimental.pallas import tpu_sc as plsc

BATCH = 16384
EMB = 64
NROWS = 1000000
IDX_CHUNK = 128
TBLK = 32768
HBLK = TBLK // 2
NBLK = (NROWS + TBLK - 1) // TBLK
NPACK = NBLK * HBLK


def _transform_block(ut_ref, it_ref, out_ref):
    eye = (lax.broadcasted_iota(jnp.int32, (EMB, EMB), 0)
           == lax.broadcasted_iota(jnp.int32, (EMB, EMB), 1)
           ).astype(jnp.bfloat16)
    xu = ut_ref[...].astype(jnp.bfloat16)
    tu = lax.dot_general(xu, eye, (((0,), (0,)), ((), ())),
                         preferred_element_type=jnp.float32)
    xi = it_ref[...].astype(jnp.bfloat16)
    ti = lax.dot_general(xi, eye, (((0,), (0,)), ((), ())),
                         preferred_element_type=jnp.float32)
    iu = lax.bitcast_convert_type(tu, jnp.int32)
    ii = lax.bitcast_convert_type(ti, jnp.int32)
    packed = (iu & jnp.int32(-65536)) | ((ii >> 16) & jnp.int32(0xFFFF))
    out_ref[:, 0:EMB] = packed[0:HBLK]
    out_ref[:, EMB:2 * EMB] = packed[HBLK:TBLK]


def _to_packed(user_emb, item_emb):
    ut = user_emb.T
    it = item_emb.T
    grid = (pl.cdiv(NROWS, TBLK),)
    return pl.pallas_call(
        _transform_block,
        grid=grid,
        compiler_params=pltpu.CompilerParams(
            vmem_limit_bytes=128 * 1024 * 1024),
        in_specs=[
            pl.BlockSpec((EMB, TBLK), lambda j: (0, j)),
            pl.BlockSpec((EMB, TBLK), lambda j: (0, j)),
        ],
        out_specs=pl.BlockSpec((HBLK, 2 * EMB), lambda j: (j, 0)),
        out_shape=jax.ShapeDtypeStruct((NPACK, 2 * EMB), jnp.int32),
    )(ut, it)


def _sc_gather(user_idx2d, item_idx2d, packed):
    info = plsc.get_sparse_core_info()
    nc, ns = info.num_cores, info.num_subcores
    nw = nc * ns
    rows_per_w = BATCH // nw
    chunks_per_w = rows_per_w // IDX_CHUNK

    mesh = plsc.VectorSubcoreMesh(core_axis_name="c", subcore_axis_name="s")

    @functools.partial(
        pl.kernel,
        mesh=mesh,
        out_type=[
            jax.ShapeDtypeStruct((BATCH // IDX_CHUNK, IDX_CHUNK, 2 * EMB),
                                 jnp.int32),
            jax.ShapeDtypeStruct((BATCH // IDX_CHUNK, IDX_CHUNK, 2 * EMB),
                                 jnp.int32),
        ],
        scratch_types=[
            pltpu.VMEM((2 * chunks_per_w, IDX_CHUNK), jnp.int32),
            pltpu.VMEM((chunks_per_w, IDX_CHUNK, 2 * EMB), jnp.int32),
            pltpu.SemaphoreType.DMA,
        ],
    )
    def gather_k(uidx_hbm, iidx_hbm, tbl_hbm, out_u, out_i,
                 idx_v, rows_v, sem):
        wid = lax.axis_index("s") * nc + lax.axis_index("c")
        crow = wid * chunks_per_w
        pltpu.sync_copy(uidx_hbm.at[pl.ds(crow, chunks_per_w)],
                        idx_v.at[pl.ds(0, chunks_per_w)])
        pltpu.sync_copy(iidx_hbm.at[pl.ds(crow, chunks_per_w)],
                        idx_v.at[pl.ds(chunks_per_w, chunks_per_w)])
        for half, out in enumerate((out_u, out_i)):
            cps = []
            for j in range(chunks_per_w):
                cps.append(pltpu.async_copy(
                    tbl_hbm.at[idx_v.at[half * chunks_per_w + j]],
                    rows_v.at[j], sem))
            for cp in cps:
                cp.wait()
            pltpu.sync_copy(rows_v, out.at[pl.ds(crow, chunks_per_w)])

    return gather_k(user_idx2d, item_idx2d, packed)


def _mlp_block(u_ref, v_ref, uh_ref, vh_ref, w1u_ref, w1i_ref, b1_ref,
               w2_ref, b2_ref, w3_ref, b3_ref, wo_ref, bo_ref, out_ref):
    usel = jnp.where(uh_ref[...] == 1, u_ref[:, EMB:], u_ref[:, :EMB])
    vsel = jnp.where(vh_ref[...] == 1, v_ref[:, EMB:], v_ref[:, :EMB])
    uval = lax.bitcast_convert_type(usel & jnp.int32(-65536),
                                    jnp.float32).astype(jnp.bfloat16)
    vval = lax.bitcast_convert_type(lax.shift_left(vsel, 16),
                                    jnp.float32).astype(jnp.bfloat16)
    x = jnp.dot(uval, w1u_ref[...], preferred_element_type=jnp.float32)
    x += jnp.dot(vval, w1i_ref[...], preferred_element_type=jnp.float32)
    x = jnp.maximum(x + b1_ref[...], 0.0).astype(jnp.bfloat16)
    x = jnp.dot(x, w2_ref[...], preferred_element_type=jnp.float32)
    x = jnp.maximum(x + b2_ref[...], 0.0).astype(jnp.bfloat16)
    x = jnp.dot(x, w3_ref[...], preferred_element_type=jnp.float32)
    x = jnp.maximum(x + b3_ref[...], 0.0)
    logit = jnp.sum(x * wo_ref[...], axis=1) + bo_ref[0, 0]
    out_ref[...] = jax.nn.sigmoid(logit)


def _pack_index(r):
    return (r // TBLK) * HBLK + (r % HBLK), (r // HBLK) % 2


def kernel(user_indices, item_indices, user_emb, item_emb,
           W1, b1, W2, b2, W3, b3, Wo, bo):
    packed = _to_packed(user_emb, item_emb)
    up, uh = _pack_index(user_indices)
    ip, ih = _pack_index(item_indices)
    uidx2d = up.reshape(BATCH // IDX_CHUNK, IDX_CHUNK)
    iidx2d = ip.reshape(BATCH // IDX_CHUNK, IDX_CHUNK)
    u_rows3, i_rows3 = _sc_gather(uidx2d, iidx2d, packed)
    u_rows = u_rows3.reshape(BATCH, 2 * EMB)
    i_rows = i_rows3.reshape(BATCH, 2 * EMB)

    w1u = W1[:, :EMB].T.astype(jnp.bfloat16)
    w1i = W1[:, EMB:].T.astype(jnp.bfloat16)
    w2t = W2.T.astype(jnp.bfloat16)
    w3t = W3.T.astype(jnp.bfloat16)
    b1r = b1.reshape(1, -1)
    b2r = b2.reshape(1, -1)
    b3r = b3.reshape(1, -1)
    wor = Wo.reshape(1, -1)
    bor = bo.reshape(1, 1)

    bb = 2048
    grid = (BATCH // bb,)
    full = lambda i: (0, 0)
    out = pl.pallas_call(
        _mlp_block,
        grid=grid,
        in_specs=[
            pl.BlockSpec((bb, 2 * EMB), lambda i: (i, 0)),
            pl.BlockSpec((bb, 2 * EMB), lambda i: (i, 0)),
            pl.BlockSpec((bb, 1), lambda i: (i, 0)),
            pl.BlockSpec((bb, 1), lambda i: (i, 0)),
            pl.BlockSpec(w1u.shape, full),
            pl.BlockSpec(w1i.shape, full),
            pl.BlockSpec(b1r.shape, full),
            pl.BlockSpec(w2t.shape, full),
            pl.BlockSpec(b2r.shape, full),
            pl.BlockSpec(w3t.shape, full),
            pl.BlockSpec(b3r.shape, full),
            pl.BlockSpec(wor.shape, full),
            pl.BlockSpec(bor.shape, full),
        ],
        out_specs=pl.BlockSpec((bb,), lambda i: (i,)),
        out_shape=jax.ShapeDtypeStruct((BATCH,), jnp.float32),
    )(u_rows, i_rows, uh.reshape(BATCH, 1), ih.reshape(BATCH, 1),
      w1u, w1i, b1r, w2t, b2r, w3t, b3r, wor, bor)
    return out


# speedup vs baseline: 4.8139x; 1.0003x over previous
import functools

import jax
import jax.numpy as jnp
from jax import lax
from jax.experimental import pallas as pl
from jax.experimental.pallas import tpu as pltpu
from jax.experimental.pallas import tpu_sc as plsc

BATCH = 16384
EMB = 64
NROWS = 1000000
IDX_CHUNK = 128
TBLK = 32768
HBLK = TBLK // 2
NBLK = (NROWS + TBLK - 1) // TBLK
NPACK = NBLK * HBLK


def _transform_block(ut_ref, it_ref, out_ref):
    eye = (lax.broadcasted_iota(jnp.int32, (EMB, EMB), 0)
           == lax.broadcasted_iota(jnp.int32, (EMB, EMB), 1)
           ).astype(jnp.bfloat16)
    xu = ut_ref[...].astype(jnp.bfloat16)
    tu = lax.dot_general(xu, eye, (((0,), (0,)), ((), ())),
                         preferred_element_type=jnp.float32)
    xi = it_ref[...].astype(jnp.bfloat16)
    ti = lax.dot_general(xi, eye, (((0,), (0,)), ((), ())),
                         preferred_element_type=jnp.float32)
    iu = lax.bitcast_convert_type(tu, jnp.int32)
    ii = lax.bitcast_convert_type(ti, jnp.int32)
    packed = (iu & jnp.int32(-65536)) | ((ii >> 16) & jnp.int32(0xFFFF))
    out_ref[:, 0:EMB] = packed[0:HBLK]
    out_ref[:, EMB:2 * EMB] = packed[HBLK:TBLK]


def _to_packed(user_emb, item_emb):
    ut = user_emb.T
    it = item_emb.T
    grid = (pl.cdiv(NROWS, TBLK),)
    return pl.pallas_call(
        _transform_block,
        grid=grid,
        compiler_params=pltpu.CompilerParams(
            vmem_limit_bytes=128 * 1024 * 1024),
        in_specs=[
            pl.BlockSpec((EMB, TBLK), lambda j: (0, j)),
            pl.BlockSpec((EMB, TBLK), lambda j: (0, j)),
        ],
        out_specs=pl.BlockSpec((HBLK, 2 * EMB), lambda j: (j, 0)),
        out_shape=jax.ShapeDtypeStruct((NPACK, 2 * EMB), jnp.int32),
    )(ut, it)


def _sc_gather(user_idx2d, item_idx2d, packed):
    info = plsc.get_sparse_core_info()
    nc, ns = info.num_cores, info.num_subcores
    nw = nc * ns
    rows_per_w = BATCH // nw
    chunks_per_w = rows_per_w // IDX_CHUNK

    mesh = plsc.VectorSubcoreMesh(core_axis_name="c", subcore_axis_name="s")

    @functools.partial(
        pl.kernel,
        mesh=mesh,
        out_type=[
            jax.ShapeDtypeStruct((BATCH // IDX_CHUNK, IDX_CHUNK, 2 * EMB),
                                 jnp.int32),
            jax.ShapeDtypeStruct((BATCH // IDX_CHUNK, IDX_CHUNK, 2 * EMB),
                                 jnp.int32),
        ],
        scratch_types=[
            pltpu.VMEM((2 * chunks_per_w, IDX_CHUNK), jnp.int32),
            pltpu.VMEM((chunks_per_w, IDX_CHUNK, 2 * EMB), jnp.int32),
            pltpu.SemaphoreType.DMA,
        ],
    )
    def gather_k(uidx_hbm, iidx_hbm, tbl_hbm, out_u, out_i,
                 idx_v, rows_v, sem):
        wid = lax.axis_index("s") * nc + lax.axis_index("c")
        crow = wid * chunks_per_w
        pltpu.sync_copy(uidx_hbm.at[pl.ds(crow, chunks_per_w)],
                        idx_v.at[pl.ds(0, chunks_per_w)])
        pltpu.sync_copy(iidx_hbm.at[pl.ds(crow, chunks_per_w)],
                        idx_v.at[pl.ds(chunks_per_w, chunks_per_w)])
        for half, out in enumerate((out_u, out_i)):
            cps = []
            for j in range(chunks_per_w):
                cps.append(pltpu.async_copy(
                    tbl_hbm.at[idx_v.at[half * chunks_per_w + j]],
                    rows_v.at[j], sem))
            for cp in cps:
                cp.wait()
            pltpu.sync_copy(rows_v, out.at[pl.ds(crow, chunks_per_w)])

    return gather_k(user_idx2d, item_idx2d, packed)


def _mlp_block(u_ref, v_ref, uh_ref, vh_ref, w1u_ref, w1i_ref, b1_ref,
               w2_ref, b2_ref, w3_ref, b3_ref, wo_ref, bo_ref, out_ref):
    usel = jnp.where(uh_ref[...] == 1, u_ref[:, EMB:], u_ref[:, :EMB])
    vsel = jnp.where(vh_ref[...] == 1, v_ref[:, EMB:], v_ref[:, :EMB])
    uval = lax.bitcast_convert_type(usel & jnp.int32(-65536),
                                    jnp.float32).astype(jnp.bfloat16)
    vval = lax.bitcast_convert_type(lax.shift_left(vsel, 16),
                                    jnp.float32).astype(jnp.bfloat16)
    x = jnp.dot(uval, w1u_ref[...], preferred_element_type=jnp.float32)
    x += jnp.dot(vval, w1i_ref[...], preferred_element_type=jnp.float32)
    x = jnp.maximum(x + b1_ref[...], 0.0).astype(jnp.bfloat16)
    x = jnp.dot(x, w2_ref[...], preferred_element_type=jnp.float32)
    x = jnp.maximum(x + b2_ref[...], 0.0).astype(jnp.bfloat16)
    x = jnp.dot(x, w3_ref[...], preferred_element_type=jnp.float32)
    x = jnp.maximum(x + b3_ref[...], 0.0)
    logit = jnp.sum(x * wo_ref[...], axis=1) + bo_ref[0, 0]
    out_ref[...] = jax.nn.sigmoid(logit)


def _pack_index(r):
    return (r // TBLK) * HBLK + (r % HBLK), (r // HBLK) % 2


def kernel(user_indices, item_indices, user_emb, item_emb,
           W1, b1, W2, b2, W3, b3, Wo, bo):
    packed = _to_packed(user_emb, item_emb)
    up, uh = _pack_index(user_indices)
    ip, ih = _pack_index(item_indices)
    uidx2d = up.reshape(BATCH // IDX_CHUNK, IDX_CHUNK)
    iidx2d = ip.reshape(BATCH // IDX_CHUNK, IDX_CHUNK)
    u_rows3, i_rows3 = _sc_gather(uidx2d, iidx2d, packed)
    u_rows = u_rows3.reshape(BATCH, 2 * EMB)
    i_rows = i_rows3.reshape(BATCH, 2 * EMB)

    w1u = W1[:, :EMB].T.astype(jnp.bfloat16)
    w1i = W1[:, EMB:].T.astype(jnp.bfloat16)
    w2t = W2.T.astype(jnp.bfloat16)
    w3t = W3.T.astype(jnp.bfloat16)
    b1r = b1.reshape(1, -1)
    b2r = b2.reshape(1, -1)
    b3r = b3.reshape(1, -1)
    wor = Wo.reshape(1, -1)
    bor = bo.reshape(1, 1)

    bb = 4096
    grid = (BATCH // bb,)
    full = lambda i: (0, 0)
    out = pl.pallas_call(
        _mlp_block,
        grid=grid,
        in_specs=[
            pl.BlockSpec((bb, 2 * EMB), lambda i: (i, 0)),
            pl.BlockSpec((bb, 2 * EMB), lambda i: (i, 0)),
            pl.BlockSpec((bb, 1), lambda i: (i, 0)),
            pl.BlockSpec((bb, 1), lambda i: (i, 0)),
            pl.BlockSpec(w1u.shape, full),
            pl.BlockSpec(w1i.shape, full),
            pl.BlockSpec(b1r.shape, full),
            pl.BlockSpec(w2t.shape, full),
            pl.BlockSpec(b2r.shape, full),
            pl.BlockSpec(w3t.shape, full),
            pl.BlockSpec(b3r.shape, full),
            pl.BlockSpec(wor.shape, full),
            pl.BlockSpec(bor.shape, full),
        ],
        out_specs=pl.BlockSpec((bb,), lambda i: (i,)),
        out_shape=jax.ShapeDtypeStruct((BATCH,), jnp.float32),
    )(u_rows, i_rows, uh.reshape(BATCH, 1), ih.reshape(BATCH, 1),
      w1u, w1i, b1r, w2t, b2r, w3t, b3r, wor, bor)
    return out
